# Initial kernel scaffold; baseline (speedup 1.0000x reference)
import jax
import jax.numpy as jnp
from jax.experimental import pallas as pl


def kernel(emb, w_ih_t, w_hh_t, b_ih, b_hh, tokens, hidden):
    raise NotImplementedError("write your pallas kernel here")


# speedup vs baseline: 1.0498x; 1.0498x over previous
import jax
import jax.numpy as jnp
from jax import lax
from jax.experimental import pallas as pl
from jax.experimental.pallas import tpu as pltpu

_T = 64
_B = 64
_D = 256
_H = 512


def _gru_fused_kernel(x_ref, h0_ref, wih_ref, whh_ref, bias_ref, bhn_ref,
                      out_ref, hT_ref, gi_ref):
    Bt = h0_ref.shape[0]
    H = _H
    pid = pl.program_id(0)

    bias = bias_ref[...]
    CH = 8
    for i in range(_T // CH):
        xc = x_ref[i * CH:(i + 1) * CH].reshape(CH * Bt, _D)
        gi_ref[i * CH * Bt:(i + 1) * CH * Bt, :] = (
            jnp.dot(xc, wih_ref[...], preferred_element_type=jnp.float32)
            + bias)

    w_hh = whh_ref[...]
    b_hn = jnp.broadcast_to(bhn_ref[...], (Bt, H))
    h = h0_ref[...]
    bidx = lax.broadcasted_iota(jnp.int32, (Bt, 1), 0) + pid * Bt

    for t in range(_T):
        gi = gi_ref[t * Bt:(t + 1) * Bt, :]
        gh = jnp.dot(h.astype(jnp.bfloat16), w_hh,
                     preferred_element_type=jnp.float32)
        r = jax.nn.sigmoid(gi[:, 0:H] + gh[:, 0:H])
        z = jax.nn.sigmoid(gi[:, H:2 * H] + gh[:, H:2 * H])
        n = jnp.tanh(gi[:, 2 * H:] + r * (gh[:, 2 * H:] + b_hn))
        h_new = (1.0 - z) * n + z * h
        mask = (bidx + t) < _T
        out_ref[t] = jnp.where(mask, h_new, 0.0)
        h = jnp.where(mask, h_new, h)
    hT_ref[...] = h


def kernel(emb, w_ih_t, w_hh_t, b_ih, b_hh, tokens, hidden):
    T, B, D, H = _T, _B, _D, _H

    x_tm = jnp.take(emb, tokens.T, axis=0).astype(jnp.bfloat16)
    bias = b_ih + jnp.concatenate(
        [b_hh[:, :2 * H], jnp.zeros((1, H), b_hh.dtype)], axis=1)
    b_hn = b_hh[:, 2 * H:]
    h0 = hidden[0]

    b_tile = B // 2
    out_tm, h_final = pl.pallas_call(
        _gru_fused_kernel,
        out_shape=(
            jax.ShapeDtypeStruct((T, B, H), jnp.float32),
            jax.ShapeDtypeStruct((B, H), jnp.float32),
        ),
        grid_spec=pltpu.PrefetchScalarGridSpec(
            num_scalar_prefetch=0,
            grid=(B // b_tile,),
            in_specs=[
                pl.BlockSpec((T, b_tile, D), lambda b: (0, b, 0)),
                pl.BlockSpec((b_tile, H), lambda b: (b, 0)),
                pl.BlockSpec((D, 3 * H), lambda b: (0, 0)),
                pl.BlockSpec((H, 3 * H), lambda b: (0, 0)),
                pl.BlockSpec((1, 3 * H), lambda b: (0, 0)),
                pl.BlockSpec((1, H), lambda b: (0, 0)),
            ],
            out_specs=[
                pl.BlockSpec((T, b_tile, H), lambda b: (0, b, 0)),
                pl.BlockSpec((b_tile, H), lambda b: (b, 0)),
            ],
            scratch_shapes=[pltpu.VMEM((T * b_tile, 3 * H), jnp.float32)],
        ),
        compiler_params=pltpu.CompilerParams(
            dimension_semantics=("parallel",)),
    )(x_tm, h0, w_ih_t.astype(jnp.bfloat16), w_hh_t.astype(jnp.bfloat16),
      bias, b_hn)

    output = jnp.transpose(out_tm, (1, 0, 2))
    return output, h_final[None]


# speedup vs baseline: 1.8037x; 1.7182x over previous
import jax
import jax.numpy as jnp
from jax import lax
from jax.experimental import pallas as pl
from jax.experimental.pallas import tpu as pltpu

_T = 64
_B = 64
_D = 256
_H = 512
_TC = 8


def _gru_fused_kernel(x_ref, h0_ref, wih_ref, whh_ref, bih_ref, bhh_ref,
                      out_ref, hT_ref,
                      wih_bf_ref, whh_bf_ref, h_ref, gi_ref):
    tau = pl.program_id(0)
    B, H, TC = _B, _H, _TC

    @pl.when(tau == 0)
    def _init():
        wih_bf_ref[...] = wih_ref[...].astype(jnp.bfloat16)
        whh_bf_ref[...] = whh_ref[...].astype(jnp.bfloat16)
        h_ref[...] = h0_ref[...]

    bhh = bhh_ref[...]
    bias = bih_ref[...] + jnp.concatenate(
        [bhh[:, :2 * H], jnp.zeros((1, H), jnp.float32)], axis=1)
    b_hn = jnp.broadcast_to(bhh[:, 2 * H:], (B, H))

    xc = x_ref[...].reshape(TC * B, _D).astype(jnp.bfloat16)
    gi_ref[...] = (
        jnp.dot(xc, wih_bf_ref[...], preferred_element_type=jnp.float32)
        + bias)

    h = h_ref[...]
    bidx = lax.broadcasted_iota(jnp.int32, (B, 1), 0)

    for k in range(TC):
        t = tau * TC + k
        gi = gi_ref[k * B:(k + 1) * B, :]
        gh = jnp.dot(h.astype(jnp.bfloat16), whh_bf_ref[...],
                     preferred_element_type=jnp.float32)
        r = jax.nn.sigmoid(gi[:, 0:H] + gh[:, 0:H])
        z = jax.nn.sigmoid(gi[:, H:2 * H] + gh[:, H:2 * H])
        n = jnp.tanh(gi[:, 2 * H:] + r * (gh[:, 2 * H:] + b_hn))
        h_new = (1.0 - z) * n + z * h
        mask = (bidx + t) < _T
        out_ref[:, k, :] = jnp.where(mask, h_new, 0.0)
        h = jnp.where(mask, h_new, h)
    h_ref[...] = h

    @pl.when(tau == _T // TC - 1)
    def _fin():
        hT_ref[...] = h


def kernel(emb, w_ih_t, w_hh_t, b_ih, b_hh, tokens, hidden):
    T, B, D, H, TC = _T, _B, _D, _H, _TC

    x_tm = jnp.take(emb, tokens.T, axis=0)
    h0 = hidden[0]

    output, h_final = pl.pallas_call(
        _gru_fused_kernel,
        out_shape=(
            jax.ShapeDtypeStruct((B, T, H), jnp.float32),
            jax.ShapeDtypeStruct((B, H), jnp.float32),
        ),
        grid_spec=pltpu.PrefetchScalarGridSpec(
            num_scalar_prefetch=0,
            grid=(T // TC,),
            in_specs=[
                pl.BlockSpec((TC, B, D), lambda t: (t, 0, 0)),
                pl.BlockSpec((B, H), lambda t: (0, 0)),
                pl.BlockSpec((D, 3 * H), lambda t: (0, 0)),
                pl.BlockSpec((H, 3 * H), lambda t: (0, 0)),
                pl.BlockSpec((1, 3 * H), lambda t: (0, 0)),
                pl.BlockSpec((1, 3 * H), lambda t: (0, 0)),
            ],
            out_specs=[
                pl.BlockSpec((B, TC, H), lambda t: (0, t, 0)),
                pl.BlockSpec((B, H), lambda t: (0, 0)),
            ],
            scratch_shapes=[
                pltpu.VMEM((D, 3 * H), jnp.bfloat16),
                pltpu.VMEM((H, 3 * H), jnp.bfloat16),
                pltpu.VMEM((B, H), jnp.float32),
                pltpu.VMEM((TC * B, 3 * H), jnp.float32),
            ],
        ),
        compiler_params=pltpu.CompilerParams(
            dimension_semantics=("arbitrary",)),
    )(x_tm, h0, w_ih_t, w_hh_t, b_ih, b_hh)

    return output, h_final[None]


# speedup vs baseline: 2.1497x; 1.1918x over previous
import jax
import jax.numpy as jnp
from jax import lax
from jax.experimental import pallas as pl
from jax.experimental.pallas import tpu as pltpu

_T = 64
_B = 64
_D = 256
_H = 512
_TC = 8
_NC = _T // _TC
_ROWS = _TC * _B


def _gru_fused_kernel(tok_ref, emb_ref, h0_ref, wih_ref, whh_ref, bih_ref,
                      bhh_ref, out_ref, hT_ref,
                      xbuf_ref, dsem, wih_bf_ref, whh_bf_ref, h_ref, gi_ref):
    tau = pl.program_id(0)
    B, H, TC = _B, _H, _TC

    def issue(chunk, slot):
        base = chunk * _ROWS
        for i in range(_ROWS):
            pltpu.make_async_copy(
                emb_ref.at[pl.ds(tok_ref[base + i], 1), :],
                xbuf_ref.at[slot, pl.ds(i, 1), :],
                dsem.at[slot]).start()

    @pl.when(tau == 0)
    def _prologue():
        issue(0, 0)

    nxt = lax.rem(tau + 1, _NC)

    @pl.when(lax.rem(tau, 2) == 0)
    def _issue_odd():
        issue(nxt, 1)

    @pl.when(lax.rem(tau, 2) == 1)
    def _issue_even():
        issue(nxt, 0)

    @pl.when(tau == 0)
    def _init():
        wih_bf_ref[...] = wih_ref[...].astype(jnp.bfloat16)
        whh_bf_ref[...] = whh_ref[...].astype(jnp.bfloat16)
        h_ref[...] = h0_ref[...]

    bhh = bhh_ref[...]
    bias = bih_ref[...] + jnp.concatenate(
        [bhh[:, :2 * H], jnp.zeros((1, H), jnp.float32)], axis=1)
    b_hn = jnp.broadcast_to(bhh[:, 2 * H:], (B, H))

    slot = lax.rem(tau, 2)
    pltpu.make_async_copy(xbuf_ref.at[slot], xbuf_ref.at[slot],
                          dsem.at[slot]).wait()
    xc = xbuf_ref[slot].astype(jnp.bfloat16)
    gi_ref[...] = (
        jnp.dot(xc, wih_bf_ref[...], preferred_element_type=jnp.float32)
        + bias)

    h = h_ref[...]
    bidx = lax.broadcasted_iota(jnp.int32, (B, 1), 0)

    for k in range(TC):
        t = tau * TC + k
        gi = gi_ref[k * B:(k + 1) * B, :]
        gh = jnp.dot(h.astype(jnp.bfloat16), whh_bf_ref[...],
                     preferred_element_type=jnp.float32)
        r = jax.nn.sigmoid(gi[:, 0:H] + gh[:, 0:H])
        z = jax.nn.sigmoid(gi[:, H:2 * H] + gh[:, H:2 * H])
        n = jnp.tanh(gi[:, 2 * H:] + r * (gh[:, 2 * H:] + b_hn))
        h_new = (1.0 - z) * n + z * h
        mask = (bidx + t) < _T
        out_ref[:, k, :] = jnp.where(mask, h_new, 0.0)
        h = jnp.where(mask, h_new, h)
    h_ref[...] = h

    @pl.when(tau == _NC - 1)
    def _fin():
        hT_ref[...] = h
        pltpu.make_async_copy(xbuf_ref.at[0], xbuf_ref.at[0],
                              dsem.at[0]).wait()


def kernel(emb, w_ih_t, w_hh_t, b_ih, b_hh, tokens, hidden):
    T, B, D, H, TC = _T, _B, _D, _H, _TC

    tokens_flat = tokens.T.reshape(T * B)
    h0 = hidden[0]

    output, h_final = pl.pallas_call(
        _gru_fused_kernel,
        out_shape=(
            jax.ShapeDtypeStruct((B, T, H), jnp.float32),
            jax.ShapeDtypeStruct((B, H), jnp.float32),
        ),
        grid_spec=pltpu.PrefetchScalarGridSpec(
            num_scalar_prefetch=1,
            grid=(T // TC,),
            in_specs=[
                pl.BlockSpec(memory_space=pltpu.MemorySpace.HBM),
                pl.BlockSpec((B, H), lambda t, *_: (0, 0)),
                pl.BlockSpec((D, 3 * H), lambda t, *_: (0, 0)),
                pl.BlockSpec((H, 3 * H), lambda t, *_: (0, 0)),
                pl.BlockSpec((1, 3 * H), lambda t, *_: (0, 0)),
                pl.BlockSpec((1, 3 * H), lambda t, *_: (0, 0)),
            ],
            out_specs=[
                pl.BlockSpec((B, TC, H), lambda t, *_: (0, t, 0)),
                pl.BlockSpec((B, H), lambda t, *_: (0, 0)),
            ],
            scratch_shapes=[
                pltpu.VMEM((2, _ROWS, D), jnp.float32),
                pltpu.SemaphoreType.DMA((2,)),
                pltpu.VMEM((D, 3 * H), jnp.bfloat16),
                pltpu.VMEM((H, 3 * H), jnp.bfloat16),
                pltpu.VMEM((B, H), jnp.float32),
                pltpu.VMEM((_ROWS, 3 * H), jnp.float32),
            ],
        ),
        compiler_params=pltpu.CompilerParams(
            dimension_semantics=("arbitrary",)),
    )(tokens_flat, emb, h0, w_ih_t, w_hh_t, b_ih, b_hh)

    return output, h_final[None]
